# baseline (speedup 1.0000x reference)
import jax
import jax.numpy as jnp
from jax import lax
from jax.experimental import pallas as pl
from jax.experimental.pallas import tpu as pltpu
from jax.experimental.pallas import tpu_sc as plsc

E = 8
TOPK = 2
C = 1024
INNER = 1024

BLK = 256
BT_R = 1024

NC, NS = 2, 16
NW = NC * NS
GCH = 40
CCH = 16


def _router_body(x_ref, wr_ref, br_ref, e0_ref, e1_ref, w0_ref, w1_ref):
    xb = x_ref[...]
    logits = lax.dot_general(
        xb, wr_ref[...], (((1,), (1,)), ((), ())),
        preferred_element_type=jnp.float32) + br_ref[...]
    p = jax.nn.softmax(logits, axis=-1)
    iota_e = lax.broadcasted_iota(jnp.int32, p.shape, 1)
    c1 = jnp.argmax(p, axis=-1)
    p1 = jnp.max(p, axis=-1)
    p_m = jnp.where(iota_e == c1[:, None], -jnp.inf, p)
    c2 = jnp.argmax(p_m, axis=-1)
    p2 = jnp.max(p_m, axis=-1)
    e0_ref[...] = c1[:, None].astype(jnp.int32)
    e1_ref[...] = c2[:, None].astype(jnp.int32)
    w0_ref[...] = p1[:, None]
    w1_ref[...] = p2[:, None]


def _sc_gather_body(x_hbm, src_hbm, xs_hbm, idx_v, rows_a, rows_b,
                    sem_a, sem_b):
    wid = lax.axis_index("s") * NC + lax.axis_index("c")
    rows_per_w = xs_hbm.shape[0] // NW
    nch = rows_per_w // GCH
    base = wid * rows_per_w
    pltpu.sync_copy(src_hbm.at[pl.ds(base, rows_per_w)], idx_v)
    bufs = (rows_a, rows_b)
    sems = (sem_a, sem_b)
    for b in range(2):
        pltpu.async_copy(
            x_hbm.at[idx_v.at[pl.ds(b * GCH, GCH)]], bufs[b], sems[b])

    def step(g, carry):
        for b in range(2):
            ch = g * 2 + b
            pltpu.make_async_copy(
                x_hbm.at[pl.ds(0, GCH)], bufs[b], sems[b]).wait()
            pltpu.sync_copy(bufs[b], xs_hbm.at[pl.ds(base + ch * GCH, GCH)])

            @pl.when(ch + 2 < nch)
            def _():
                pltpu.async_copy(
                    x_hbm.at[idx_v.at[pl.ds((ch + 2) * GCH, GCH)]],
                    bufs[b], sems[b])
        return carry

    lax.fori_loop(0, nch // 2, step, 0)


def _mm_body(be_ref, xs_ref, w1_ref, b1_ref, wg_ref, bg_ref, w2_ref, b2_ref,
             ws_ref, ys_ref):
    xb = xs_ref[...].astype(jnp.bfloat16)
    h1 = lax.dot_general(
        xb, w1_ref[0], (((1,), (1,)), ((), ())),
        preferred_element_type=jnp.float32) + b1_ref[0]
    hg = lax.dot_general(
        xb, wg_ref[0], (((1,), (1,)), ((), ())),
        preferred_element_type=jnp.float32) + bg_ref[0]
    h = ((h1 * jax.nn.sigmoid(h1)) * hg).astype(jnp.bfloat16)
    eo = lax.dot_general(
        h, w2_ref[0], (((1,), (1,)), ((), ())),
        preferred_element_type=jnp.float32) + b2_ref[0]
    ys_ref[...] = eo * ws_ref[...]


def _sc_combine_body(ys_hbm, pos_hbm, out_hbm, pidx_v, in_a, in_b, out_v,
                     sem_a, sem_b):
    wid = lax.axis_index("s") * NC + lax.axis_index("c")
    tok_per_w = out_hbm.shape[0] // NW
    nch = tok_per_w // CCH
    tbase = wid * tok_per_w
    pltpu.sync_copy(pos_hbm.at[pl.ds(tbase * 2, tok_per_w * 2)], pidx_v)
    bufs = (in_a, in_b)
    sems = (sem_a, sem_b)
    for b in range(2):
        pltpu.async_copy(
            ys_hbm.at[pidx_v.at[pl.ds(b * 2 * CCH, 2 * CCH)]], bufs[b],
            sems[b])

    def step(g, carry):
        for b in range(2):
            ch = g * 2 + b
            pltpu.make_async_copy(
                ys_hbm.at[pl.ds(0, 2 * CCH)], bufs[b], sems[b]).wait()
            buf = bufs[b]

            def row_add(r, c2):
                for k in range(C // 16):
                    sl = pl.ds(k * 16, 16)
                    out_v[r, sl] = buf[2 * r, sl] + buf[2 * r + 1, sl]
                return c2

            lax.fori_loop(0, CCH, row_add, 0)
            pltpu.sync_copy(out_v, out_hbm.at[pl.ds(tbase + ch * CCH, CCH)])

            @pl.when(ch + 2 < nch)
            def _():
                pltpu.async_copy(
                    ys_hbm.at[pidx_v.at[pl.ds((ch + 2) * 2 * CCH, 2 * CCH)]],
                    bufs[b], sems[b])
        return carry

    lax.fori_loop(0, nch // 2, step, 0)


def _meta(e0h, e1h, w0h, w1h, tok_base, nb):
    nh = e0h.shape[0]
    ph = nh * TOPK
    npad = nb * BLK
    e_flat = jnp.concatenate([e0h, e1h], axis=1).reshape(ph)
    w_flat = jnp.concatenate([w0h, w1h], axis=1).reshape(ph)
    oh = (e_flat[:, None] == jnp.arange(E, dtype=jnp.int32)[None, :])
    csum = jnp.cumsum(oh.astype(jnp.int32), axis=0)
    counts = csum[-1]
    rank = jnp.take_along_axis(csum, e_flat[:, None], axis=1)[:, 0] - 1
    pc = ((counts + BLK - 1) // BLK) * BLK
    ends = jnp.cumsum(pc)
    po = ends - pc
    pos = (po[e_flat] + rank).astype(jnp.int32)
    src = jnp.zeros((npad,), jnp.int32).at[pos].set(
        tok_base + jnp.arange(ph, dtype=jnp.int32) // TOPK)
    wsort = jnp.zeros((npad,), jnp.float32).at[pos].set(w_flat)
    bstarts = jnp.arange(nb, dtype=jnp.int32) * BLK
    be = jnp.minimum(
        jnp.searchsorted(ends, bstarts, side='right').astype(jnp.int32),
        E - 1)
    return src, wsort, be, pos


_SC_MESH = dict(core_axis_name="c", subcore_axis_name="s",
                num_cores=NC, num_subcores=NS)


def _gather_call(x2, src, npad):
    return pl.kernel(
        _sc_gather_body,
        out_type=jax.ShapeDtypeStruct((npad, C), jnp.float32),
        mesh=plsc.VectorSubcoreMesh(**_SC_MESH),
        scratch_types=[
            pltpu.VMEM((npad // NW,), jnp.int32),
            pltpu.VMEM((GCH, C), jnp.float32),
            pltpu.VMEM((GCH, C), jnp.float32),
            pltpu.SemaphoreType.DMA,
            pltpu.SemaphoreType.DMA,
        ],
    )(x2, src)


def _mm_call(be, xs, wsort, W1b, b1r, Wgb, bgr, W2b, b2r, nb):
    npad = nb * BLK
    grid_spec = pltpu.PrefetchScalarGridSpec(
        num_scalar_prefetch=1,
        grid=(nb,),
        in_specs=[
            pl.BlockSpec((BLK, C), lambda i, be: (i, 0)),
            pl.BlockSpec((1, INNER, C), lambda i, be: (be[i], 0, 0)),
            pl.BlockSpec((1, 1, INNER), lambda i, be: (be[i], 0, 0)),
            pl.BlockSpec((1, INNER, C), lambda i, be: (be[i], 0, 0)),
            pl.BlockSpec((1, 1, INNER), lambda i, be: (be[i], 0, 0)),
            pl.BlockSpec((1, C, INNER), lambda i, be: (be[i], 0, 0)),
            pl.BlockSpec((1, 1, C), lambda i, be: (be[i], 0, 0)),
            pl.BlockSpec((BLK, 1), lambda i, be: (i, 0)),
        ],
        out_specs=pl.BlockSpec((BLK, C), lambda i, be: (i, 0)),
    )
    return pl.pallas_call(
        _mm_body,
        grid_spec=grid_spec,
        out_shape=jax.ShapeDtypeStruct((npad, C), jnp.float32),
        compiler_params=pltpu.CompilerParams(
            dimension_semantics=("arbitrary",)),
    )(be, xs, W1b, b1r, Wgb, bgr, W2b, b2r, wsort.reshape(npad, 1))


def _combine_call(ys, pos, nh):
    return pl.kernel(
        _sc_combine_body,
        out_type=jax.ShapeDtypeStruct((nh, C), jnp.float32),
        mesh=plsc.VectorSubcoreMesh(**_SC_MESH),
        scratch_types=[
            pltpu.VMEM((2 * nh // NW,), jnp.int32),
            pltpu.VMEM((2 * CCH, C), jnp.float32),
            pltpu.VMEM((2 * CCH, C), jnp.float32),
            pltpu.VMEM((CCH, C), jnp.float32),
            pltpu.SemaphoreType.DMA,
            pltpu.SemaphoreType.DMA,
        ],
    )(ys, pos)


@jax.jit
def kernel(x, Wr, br, W1, b1, Wg, bg, W2, b2):
    B, T, _ = x.shape
    N = B * T
    NH = N // 2
    NBH = NH * TOPK // BLK + E
    x2 = x.reshape(N, C)

    e0, e1, w0, w1 = pl.pallas_call(
        _router_body,
        grid=(N // BT_R,),
        in_specs=[
            pl.BlockSpec((BT_R, C), lambda i: (i, 0)),
            pl.BlockSpec((E, C), lambda i: (0, 0)),
            pl.BlockSpec((1, E), lambda i: (0, 0)),
        ],
        out_specs=[
            pl.BlockSpec((BT_R, 1), lambda i: (i, 0)),
            pl.BlockSpec((BT_R, 1), lambda i: (i, 0)),
            pl.BlockSpec((BT_R, 1), lambda i: (i, 0)),
            pl.BlockSpec((BT_R, 1), lambda i: (i, 0)),
        ],
        out_shape=[
            jax.ShapeDtypeStruct((N, 1), jnp.int32),
            jax.ShapeDtypeStruct((N, 1), jnp.int32),
            jax.ShapeDtypeStruct((N, 1), jnp.float32),
            jax.ShapeDtypeStruct((N, 1), jnp.float32),
        ],
    )(x2, Wr, br.reshape(1, E))

    W1b = W1.astype(jnp.bfloat16)
    Wgb = Wg.astype(jnp.bfloat16)
    W2b = W2.astype(jnp.bfloat16)
    b1r = b1.reshape(E, 1, INNER)
    bgr = bg.reshape(E, 1, INNER)
    b2r = b2.reshape(E, 1, C)

    src0, ws0, be0, pos0 = _meta(e0[:NH], e1[:NH], w0[:NH], w1[:NH], 0, NBH)
    src1, ws1, be1, pos1 = _meta(e0[NH:], e1[NH:], w0[NH:], w1[NH:], NH, NBH)

    xs0 = _gather_call(x2, src0, NBH * BLK)
    ys0 = _mm_call(be0, xs0, ws0, W1b, b1r, Wgb, bgr, W2b, b2r, NBH)
    xs1 = _gather_call(x2, src1, NBH * BLK)
    out0 = _combine_call(ys0, pos0, NH)
    ys1 = _mm_call(be1, xs1, ws1, W1b, b1r, Wgb, bgr, W2b, b2r, NBH)
    out1 = _combine_call(ys1, pos1, NH)

    return jnp.concatenate([out0, out1], axis=0).reshape(B, T, C)


# speedup vs baseline: 1.4933x; 1.4933x over previous
import jax
import jax.numpy as jnp
from jax import lax
from jax.experimental import pallas as pl
from jax.experimental.pallas import tpu as pltpu
from jax.experimental.pallas import tpu_sc as plsc

E = 8
TOPK = 2
C = 1024
INNER = 1024

BLK = 256
BT_R = 1024

NC, NS = 2, 16
NW = NC * NS
GCH = 40
CCH = 16


def _router_body(x_ref, wr_ref, br_ref, e0_ref, e1_ref, w0_ref, w1_ref):
    xb = x_ref[...]
    logits = lax.dot_general(
        xb, wr_ref[...], (((1,), (1,)), ((), ())),
        preferred_element_type=jnp.float32) + br_ref[...]
    p = jax.nn.softmax(logits, axis=-1)
    iota_e = lax.broadcasted_iota(jnp.int32, p.shape, 1)
    c1 = jnp.argmax(p, axis=-1)
    p1 = jnp.max(p, axis=-1)
    p_m = jnp.where(iota_e == c1[:, None], -jnp.inf, p)
    c2 = jnp.argmax(p_m, axis=-1)
    p2 = jnp.max(p_m, axis=-1)
    e0_ref[...] = c1[:, None].astype(jnp.int32)
    e1_ref[...] = c2[:, None].astype(jnp.int32)
    w0_ref[...] = p1[:, None]
    w1_ref[...] = p2[:, None]


def _sc_gather_body(x_hbm, src_hbm, xs_hbm, idx_v, rows_a, rows_b,
                    sem_a, sem_b):
    wid = lax.axis_index("s") * NC + lax.axis_index("c")
    rows_per_w = xs_hbm.shape[0] // NW
    nch = rows_per_w // GCH
    base = wid * rows_per_w
    pltpu.sync_copy(src_hbm.at[pl.ds(base, rows_per_w)], idx_v)
    bufs = (rows_a, rows_b)
    sems = (sem_a, sem_b)
    for b in range(2):
        pltpu.async_copy(
            x_hbm.at[idx_v.at[pl.ds(b * GCH, GCH)]], bufs[b], sems[b])

    def step(g, carry):
        for b in range(2):
            ch = g * 2 + b
            pltpu.make_async_copy(
                x_hbm.at[pl.ds(0, GCH)], bufs[b], sems[b]).wait()
            pltpu.sync_copy(bufs[b], xs_hbm.at[pl.ds(base + ch * GCH, GCH)])

            @pl.when(ch + 2 < nch)
            def _():
                pltpu.async_copy(
                    x_hbm.at[idx_v.at[pl.ds((ch + 2) * GCH, GCH)]],
                    bufs[b], sems[b])
        return carry

    lax.fori_loop(0, nch // 2, step, 0)


def _mm_body(be_ref, xs_ref, w1_ref, b1_ref, wg_ref, bg_ref, w2_ref, b2_ref,
             ws_ref, ys_ref):
    xb = xs_ref[...].astype(jnp.bfloat16)
    h1 = lax.dot_general(
        xb, w1_ref[0].astype(jnp.bfloat16), (((1,), (1,)), ((), ())),
        preferred_element_type=jnp.float32) + b1_ref[0]
    hg = lax.dot_general(
        xb, wg_ref[0].astype(jnp.bfloat16), (((1,), (1,)), ((), ())),
        preferred_element_type=jnp.float32) + bg_ref[0]
    h = ((h1 * jax.nn.sigmoid(h1)) * hg).astype(jnp.bfloat16)
    eo = lax.dot_general(
        h, w2_ref[0].astype(jnp.bfloat16), (((1,), (1,)), ((), ())),
        preferred_element_type=jnp.float32) + b2_ref[0]
    ys_ref[...] = eo * ws_ref[...]


def _sc_combine_body(ys_hbm, pos_hbm, out_hbm, pidx_v, in_a, in_b, out_v,
                     sem_a, sem_b):
    wid = lax.axis_index("s") * NC + lax.axis_index("c")
    tok_per_w = out_hbm.shape[0] // NW
    nch = tok_per_w // CCH
    tbase = wid * tok_per_w
    pltpu.sync_copy(pos_hbm.at[pl.ds(tbase * 2, tok_per_w * 2)], pidx_v)
    bufs = (in_a, in_b)
    sems = (sem_a, sem_b)
    for b in range(2):
        pltpu.async_copy(
            ys_hbm.at[pidx_v.at[pl.ds(b * 2 * CCH, 2 * CCH)]], bufs[b],
            sems[b])

    def step(g, carry):
        for b in range(2):
            ch = g * 2 + b
            pltpu.make_async_copy(
                ys_hbm.at[pl.ds(0, 2 * CCH)], bufs[b], sems[b]).wait()
            buf = bufs[b]

            def row_add(r, c2):
                for k in range(C // 16):
                    sl = pl.ds(k * 16, 16)
                    out_v[r, sl] = buf[2 * r, sl] + buf[2 * r + 1, sl]
                return c2

            lax.fori_loop(0, CCH, row_add, 0)
            pltpu.sync_copy(out_v, out_hbm.at[pl.ds(tbase + ch * CCH, CCH)])

            @pl.when(ch + 2 < nch)
            def _():
                pltpu.async_copy(
                    ys_hbm.at[pidx_v.at[pl.ds((ch + 2) * 2 * CCH, 2 * CCH)]],
                    bufs[b], sems[b])
        return carry

    lax.fori_loop(0, nch // 2, step, 0)


def _meta(e0h, e1h, w0h, w1h, tok_base, nb):
    nh = e0h.shape[0]
    ph = nh * TOPK
    npad = nb * BLK
    e_flat = jnp.concatenate([e0h, e1h], axis=1).reshape(ph)
    w_flat = jnp.concatenate([w0h, w1h], axis=1).reshape(ph)
    oh = (e_flat[:, None] == jnp.arange(E, dtype=jnp.int32)[None, :])
    csum = jnp.cumsum(oh.astype(jnp.int32), axis=0)
    counts = csum[-1]
    rank = jnp.take_along_axis(csum, e_flat[:, None], axis=1)[:, 0] - 1
    pc = ((counts + BLK - 1) // BLK) * BLK
    ends = jnp.cumsum(pc)
    po = ends - pc
    pos = (po[e_flat] + rank).astype(jnp.int32)
    toks = (tok_base + jnp.arange(ph, dtype=jnp.int32) // TOPK)
    init = jnp.stack(
        [(jnp.arange(npad, dtype=jnp.int32) % nh + tok_base)
         .astype(jnp.float32),
         jnp.zeros((npad,), jnp.float32)], axis=1)
    scat = init.at[pos].set(
        jnp.stack([toks.astype(jnp.float32), w_flat], axis=1))
    src = scat[:, 0].astype(jnp.int32)
    wsort = scat[:, 1]
    bstarts = jnp.arange(nb, dtype=jnp.int32) * BLK
    be = jnp.minimum(
        jnp.sum((bstarts[:, None] >= ends[None, :]).astype(jnp.int32),
                axis=1), E - 1)
    return src, wsort, be, pos


_SC_MESH = dict(core_axis_name="c", subcore_axis_name="s",
                num_cores=NC, num_subcores=NS)


def _gather_call(x2, src, npad):
    return pl.kernel(
        _sc_gather_body,
        out_type=jax.ShapeDtypeStruct((npad, C), jnp.float32),
        mesh=plsc.VectorSubcoreMesh(**_SC_MESH),
        scratch_types=[
            pltpu.VMEM((npad // NW,), jnp.int32),
            pltpu.VMEM((GCH, C), jnp.float32),
            pltpu.VMEM((GCH, C), jnp.float32),
            pltpu.SemaphoreType.DMA,
            pltpu.SemaphoreType.DMA,
        ],
    )(x2, src)


def _mm_call(be, xs, wsort, W1b, b1r, Wgb, bgr, W2b, b2r, nb):
    npad = nb * BLK
    grid_spec = pltpu.PrefetchScalarGridSpec(
        num_scalar_prefetch=1,
        grid=(nb,),
        in_specs=[
            pl.BlockSpec((BLK, C), lambda i, be: (i, 0)),
            pl.BlockSpec((1, INNER, C), lambda i, be: (be[i], 0, 0)),
            pl.BlockSpec((1, 1, INNER), lambda i, be: (be[i], 0, 0)),
            pl.BlockSpec((1, INNER, C), lambda i, be: (be[i], 0, 0)),
            pl.BlockSpec((1, 1, INNER), lambda i, be: (be[i], 0, 0)),
            pl.BlockSpec((1, C, INNER), lambda i, be: (be[i], 0, 0)),
            pl.BlockSpec((1, 1, C), lambda i, be: (be[i], 0, 0)),
            pl.BlockSpec((BLK, 1), lambda i, be: (i, 0)),
        ],
        out_specs=pl.BlockSpec((BLK, C), lambda i, be: (i, 0)),
    )
    return pl.pallas_call(
        _mm_body,
        grid_spec=grid_spec,
        out_shape=jax.ShapeDtypeStruct((npad, C), jnp.float32),
        compiler_params=pltpu.CompilerParams(
            dimension_semantics=("arbitrary",)),
    )(be, xs, W1b, b1r, Wgb, bgr, W2b, b2r, wsort.reshape(npad, 1))


def _combine_call(ys, pos, nh):
    return pl.kernel(
        _sc_combine_body,
        out_type=jax.ShapeDtypeStruct((nh, C), jnp.float32),
        mesh=plsc.VectorSubcoreMesh(**_SC_MESH),
        scratch_types=[
            pltpu.VMEM((2 * nh // NW,), jnp.int32),
            pltpu.VMEM((2 * CCH, C), jnp.float32),
            pltpu.VMEM((2 * CCH, C), jnp.float32),
            pltpu.VMEM((CCH, C), jnp.float32),
            pltpu.SemaphoreType.DMA,
            pltpu.SemaphoreType.DMA,
        ],
    )(ys, pos)


@jax.jit
def kernel(x, Wr, br, W1, b1, Wg, bg, W2, b2):
    B, T, _ = x.shape
    N = B * T
    NH = N // 2
    NBH = NH * TOPK // BLK + E
    x2 = x.reshape(N, C)

    e0, e1, w0, w1 = pl.pallas_call(
        _router_body,
        grid=(N // BT_R,),
        in_specs=[
            pl.BlockSpec((BT_R, C), lambda i: (i, 0)),
            pl.BlockSpec((E, C), lambda i: (0, 0)),
            pl.BlockSpec((1, E), lambda i: (0, 0)),
        ],
        out_specs=[
            pl.BlockSpec((BT_R, 1), lambda i: (i, 0)),
            pl.BlockSpec((BT_R, 1), lambda i: (i, 0)),
            pl.BlockSpec((BT_R, 1), lambda i: (i, 0)),
            pl.BlockSpec((BT_R, 1), lambda i: (i, 0)),
        ],
        out_shape=[
            jax.ShapeDtypeStruct((N, 1), jnp.int32),
            jax.ShapeDtypeStruct((N, 1), jnp.int32),
            jax.ShapeDtypeStruct((N, 1), jnp.float32),
            jax.ShapeDtypeStruct((N, 1), jnp.float32),
        ],
    )(x2, Wr, br.reshape(1, E))

    W1b = W1
    Wgb = Wg
    W2b = W2
    b1r = b1.reshape(E, 1, INNER)
    bgr = bg.reshape(E, 1, INNER)
    b2r = b2.reshape(E, 1, C)

    src0, ws0, be0, pos0 = _meta(e0[:NH], e1[:NH], w0[:NH], w1[:NH], 0, NBH)
    src1, ws1, be1, pos1 = _meta(e0[NH:], e1[NH:], w0[NH:], w1[NH:], NH, NBH)

    xs0 = _gather_call(x2, src0, NBH * BLK)
    ys0 = _mm_call(be0, xs0, ws0, W1b, b1r, Wgb, bgr, W2b, b2r, NBH)
    xs1 = _gather_call(x2, src1, NBH * BLK)
    out0 = _combine_call(ys0, pos0, NH)
    ys1 = _mm_call(be1, xs1, ws1, W1b, b1r, Wgb, bgr, W2b, b2r, NBH)
    out1 = _combine_call(ys1, pos1, NH)

    return jnp.concatenate([out0, out1], axis=0).reshape(B, T, C)
